# baseline (speedup 1.0000x reference)
import jax
import jax.numpy as jnp
from jax.experimental import pallas as pl
from jax.experimental.pallas import tpu as pltpu

N_TOK = 16384
N_EMB = 8192
DIM = 256
TM = 128
COMMIT = 0.25


def _argmin_body(x_ref, wt_ref, idx_ref, b_ref, m_scr, a_scr):
    i = pl.program_id(0)
    nsteps = pl.num_programs(0) - 1

    @pl.when(i == 0)
    def _():
        wt = wt_ref[...]
        b_ref[...] = jnp.sum(wt * wt, axis=0, keepdims=True)

    @pl.when(i < nsteps)
    def _():
        x = x_ref[...]
        a_scr[i % 2] = jnp.sum(x * x, axis=1, keepdims=True)
        m_scr[i % 2] = jax.lax.dot_general(
            x, wt_ref[...], (((1,), (0,)), ((), ())),
            preferred_element_type=jnp.float32,
        )

    @pl.when(i > 0)
    def _():
        m = m_scr[(i - 1) % 2]
        a = a_scr[(i - 1) % 2]
        d = (a + b_ref[...]) - 2.0 * m
        rowmin = jnp.min(d, axis=1, keepdims=True)
        ids = jax.lax.broadcasted_iota(jnp.int32, d.shape, 1)
        idx = jnp.min(jnp.where(d == rowmin, ids, d.shape[1]), axis=1)
        idx_ref[...] = idx[:, None]


def _argmin_call(x, wt):
    n, dim = x.shape
    k = wt.shape[1]
    nm1 = n // TM - 1
    return pl.pallas_call(
        _argmin_body,
        grid=(n // TM + 1,),
        in_specs=[
            pl.BlockSpec((TM, dim), lambda i: (jnp.minimum(i, nm1), 0)),
            pl.BlockSpec((dim, k), lambda i: (0, 0)),
        ],
        out_specs=pl.BlockSpec((TM, 1), lambda i: (jnp.maximum(i - 1, 0), 0)),
        out_shape=jax.ShapeDtypeStruct((n, 1), jnp.int32),
        scratch_shapes=[
            pltpu.VMEM((1, k), jnp.float32),
            pltpu.VMEM((2, TM, k), jnp.float32),
            pltpu.VMEM((2, TM, 1), jnp.float32),
        ],
    )(x, wt)


def kernel(inputs, W):
    encoding_indices = _argmin_call(inputs, W.T)
    quantized = jnp.take(W, encoding_indices[:, 0], axis=0)
    q_loss = jnp.mean((quantized - inputs) ** 2)
    e_loss = jnp.mean((quantized - inputs) ** 2)
    vq_loss = q_loss + COMMIT * e_loss
    quantized_st = inputs + (quantized - inputs)
    return (quantized_st, vq_loss, encoding_indices)
